# Initial kernel scaffold; baseline (speedup 1.0000x reference)
import jax
import jax.numpy as jnp
from jax.experimental import pallas as pl


def kernel(edge_index, edge_weight, user_emb, item_emb):
    raise NotImplementedError("write your pallas kernel here")


# speedup vs baseline: 1.4732x; 1.4732x over previous
import functools

import jax
import jax.numpy as jnp
from jax import lax
from jax.experimental import pallas as pl
from jax.experimental.pallas import tpu as pltpu
from jax.experimental.pallas import tpu_sc as plsc

N_USERS = 25000
N_ITEMS = 25000
N_NODES = N_USERS + N_ITEMS
D = 64
H = 32
E = 800000
NL = 3

NT = 16
SUB = 128
NSUB = 4
BLK = NSUB * SUB
GPS = SUB // 16
BPT = 98
EP = NT * BPT * BLK
NBLK = NT * BPT
RPT = N_NODES // NT
ZCH = 200


def _prop_body(emb0, colb, rowb, wb, e1, e2, e3,
               colv, rowv, wv, rowsbuf, zbuf, acc, gsem, ssem):
  c = lax.axis_index("c")
  s = lax.axis_index("s")
  iota16 = lax.iota(jnp.int32, 16)
  zero16 = jnp.zeros((16,), jnp.float32)
  coff = c * N_NODES

  @pl.loop(0, ZCH)
  def _(r):
    zbuf[r, pl.ds(0, 16)] = zero16
    zbuf[r, pl.ds(16, 16)] = zero16

  def zero_acc():
    for k in range(15):
      pltpu.sync_copy(zbuf, acc.at[pl.ds(s * RPT + k * ZCH, ZCH)])
    pltpu.sync_copy(zbuf.at[pl.ds(0, 125)],
                    acc.at[pl.ds(s * RPT + 15 * ZCH, 125)])

  zero_acc()
  plsc.subcore_barrier()

  outs = (e1, e2, e3)
  for layer in range(NL):
    src = emb0 if layer == 0 else outs[layer - 1]
    dst = outs[layer]

    @pl.loop(0, BPT)
    def _(b):
      g = s * BPT + b
      pltpu.sync_copy(colb.at[g], colv)
      pltpu.sync_copy(rowb.at[g], rowv)
      pltpu.sync_copy(wb.at[g], wv)
      for j in range(NSUB):
        for q in range(SUB // 16):
          sl = (j, pl.ds(q * 16, 16))
          colv[sl] = colv[sl] + coff
      gd = [pltpu.async_copy(src.at[colv.at[j]], rowsbuf.at[j], gsem)
            for j in range(NSUB)]
      for d_ in gd:
        d_.wait()

      @pl.loop(0, BLK // 16)
      def _(g16):
        jj = jnp.full((16,), g16 // GPS, jnp.int32)
        ii = iota16 + (g16 % GPS) * 16
        w16 = wv[pl.ds(g16 * 16, 16)]
        for d in range(H):
          dd = jnp.full((16,), d, jnp.int32)
          v = plsc.load_gather(rowsbuf, [jj, ii, dd])
          plsc.store_scatter(rowsbuf, [jj, ii, dd], v * w16)

      sd = [pltpu.async_copy(rowsbuf.at[j], acc.at[rowv.at[j]], ssem, add=True)
            for j in range(NSUB)]
      for d_ in sd:
        d_.wait()

    plsc.subcore_barrier()
    pltpu.sync_copy(acc.at[pl.ds(s * RPT, RPT)],
                    dst.at[pl.ds(coff + s * RPT, RPT)])
    if layer < NL - 1:
      zero_acc()
    plsc.subcore_barrier()


_prop = pl.kernel(
    _prop_body,
    out_type=(jax.ShapeDtypeStruct((2 * N_NODES, H), jnp.float32),) * 3,
    mesh=plsc.VectorSubcoreMesh(core_axis_name="c", subcore_axis_name="s"),
    scratch_types=[
        pltpu.VMEM((NSUB, SUB), jnp.int32),
        pltpu.VMEM((NSUB, SUB), jnp.int32),
        pltpu.VMEM((BLK,), jnp.float32),
        pltpu.VMEM((NSUB, SUB, H), jnp.float32),
        pltpu.VMEM((ZCH, H), jnp.float32),
        pltpu.VMEM_SHARED((N_NODES, H), jnp.float32),
        pltpu.SemaphoreType.DMA,
        pltpu.SemaphoreType.DMA,
    ],
    compiler_params=pltpu.CompilerParams(
        use_tc_tiling_on_sc=False, needs_layout_passes=False),
)


def _mean_body(a, b, c, d, o):
  o[...] = 0.25 * (a[...] + b[...] + c[...] + d[...])


_MROWS = 2 * N_NODES * H // 128
_MB = 1000

_mean = pl.pallas_call(
    _mean_body,
    grid=(_MROWS // _MB,),
    in_specs=[pl.BlockSpec((_MB, 128), lambda i: (i, 0))] * 4,
    out_specs=pl.BlockSpec((_MB, 128), lambda i: (i, 0)),
    out_shape=jax.ShapeDtypeStruct((_MROWS, 128), jnp.float32),
)


@jax.jit
def kernel(edge_index, edge_weight, user_emb, item_emb):
  all_emb = jnp.concatenate([user_emb, item_emb], axis=0)
  emb0 = jnp.concatenate([all_emb[:, :H], all_emb[:, H:]], axis=0)
  pad = EP - E
  col = jnp.concatenate([edge_index[1], jnp.zeros((pad,), jnp.int32)])
  row = jnp.concatenate([edge_index[0], jnp.zeros((pad,), jnp.int32)])
  w = jnp.concatenate([edge_weight, jnp.zeros((pad,), jnp.float32)])
  colb = col.reshape(NBLK, NSUB, SUB)
  rowb = row.reshape(NBLK, NSUB, SUB)
  wb = w.reshape(NBLK, BLK)
  e1, e2, e3 = _prop(emb0, colb, rowb, wb)
  m = _mean(emb0.reshape(_MROWS, 128), e1.reshape(_MROWS, 128),
            e2.reshape(_MROWS, 128), e3.reshape(_MROWS, 128))
  m = m.reshape(2 * N_NODES, H)
  final = jnp.concatenate([m[:N_NODES], m[N_NODES:]], axis=1)
  return final[:N_USERS], final[N_USERS:]


# speedup vs baseline: 7.7176x; 5.2386x over previous
import functools

import jax
import jax.numpy as jnp
from jax import lax
from jax.experimental import pallas as pl
from jax.experimental.pallas import tpu as pltpu
from jax.experimental.pallas import tpu_sc as plsc

N_USERS = 25000
N_ITEMS = 25000
N_NODES = N_USERS + N_ITEMS
D = 64
H = 32
E = 800000
NL = 3

NT = 16
SUB = 128
NSUB = 4
BLK = NSUB * SUB
GPS = SUB // 16
BPT = 98
EP = NT * BPT * BLK
NBLK = NT * BPT
RPT = N_NODES // NT
ZCH = 200


def _prop_body(emb0, colb, rowb, wb, e1, e2, e3,
               colv, rowv, wv, rowsbuf, zbuf, acc, gsem, ssem):
  c = lax.axis_index("c")
  s = lax.axis_index("s")
  iota16 = lax.iota(jnp.int32, 16)
  zero16 = jnp.zeros((16,), jnp.float32)
  coff = c * N_NODES

  @pl.loop(0, ZCH)
  def _(r):
    zbuf[r, pl.ds(0, 16)] = zero16
    zbuf[r, pl.ds(16, 16)] = zero16

  def zero_acc():
    for k in range(15):
      pltpu.sync_copy(zbuf, acc.at[pl.ds(s * RPT + k * ZCH, ZCH)])
    pltpu.sync_copy(zbuf.at[pl.ds(0, 125)],
                    acc.at[pl.ds(s * RPT + 15 * ZCH, 125)])

  zero_acc()
  plsc.subcore_barrier()

  outs = (e1, e2, e3)
  for layer in range(NL):
    src = emb0 if layer == 0 else outs[layer - 1]
    dst = outs[layer]

    @pl.loop(0, BPT)
    def _(b):
      g = s * BPT + b
      pltpu.sync_copy(colb.at[g], colv)
      pltpu.sync_copy(rowb.at[g], rowv)
      pltpu.sync_copy(wb.at[g], wv)
      for j in range(NSUB):
        for q in range(SUB // 16):
          sl = (j, pl.ds(q * 16, 16))
          colv[sl] = colv[sl] + coff
      gd = [pltpu.async_copy(src.at[colv.at[j]], rowsbuf.at[j], gsem)
            for j in range(NSUB)]
      sd = []
      for j in range(NSUB):
        gd[j].wait()

        @pl.loop(0, GPS)
        def _(g, j=j):
          w16 = wv[pl.ds(j * SUB + g * 16, 16)]
          for k in range(16):
            wk = lax.broadcast_in_dim(w16[k], (16,), ())
            i = g * 16 + k
            lo = (j, i, pl.ds(0, 16))
            hi = (j, i, pl.ds(16, 16))
            rowsbuf[lo] = rowsbuf[lo] * wk
            rowsbuf[hi] = rowsbuf[hi] * wk

        sd.append(pltpu.async_copy(rowsbuf.at[j], acc.at[rowv.at[j]],
                                   ssem, add=True))
      for d_ in sd:
        d_.wait()

    plsc.subcore_barrier()
    pltpu.sync_copy(acc.at[pl.ds(s * RPT, RPT)],
                    dst.at[pl.ds(coff + s * RPT, RPT)])
    if layer < NL - 1:
      zero_acc()
    plsc.subcore_barrier()


_prop = pl.kernel(
    _prop_body,
    out_type=(jax.ShapeDtypeStruct((2 * N_NODES, H), jnp.float32),) * 3,
    mesh=plsc.VectorSubcoreMesh(core_axis_name="c", subcore_axis_name="s"),
    scratch_types=[
        pltpu.VMEM((NSUB, SUB), jnp.int32),
        pltpu.VMEM((NSUB, SUB), jnp.int32),
        pltpu.VMEM((BLK,), jnp.float32),
        pltpu.VMEM((NSUB, SUB, H), jnp.float32),
        pltpu.VMEM((ZCH, H), jnp.float32),
        pltpu.VMEM_SHARED((N_NODES, H), jnp.float32),
        pltpu.SemaphoreType.DMA,
        pltpu.SemaphoreType.DMA,
    ],
    compiler_params=pltpu.CompilerParams(
        use_tc_tiling_on_sc=False, needs_layout_passes=False),
)


def _mean_body(a, b, c, d, o):
  o[...] = 0.25 * (a[...] + b[...] + c[...] + d[...])


_MROWS = 2 * N_NODES * H // 128
_MB = 1000

_mean = pl.pallas_call(
    _mean_body,
    grid=(_MROWS // _MB,),
    in_specs=[pl.BlockSpec((_MB, 128), lambda i: (i, 0))] * 4,
    out_specs=pl.BlockSpec((_MB, 128), lambda i: (i, 0)),
    out_shape=jax.ShapeDtypeStruct((_MROWS, 128), jnp.float32),
)


@jax.jit
def kernel(edge_index, edge_weight, user_emb, item_emb):
  all_emb = jnp.concatenate([user_emb, item_emb], axis=0)
  emb0 = jnp.concatenate([all_emb[:, :H], all_emb[:, H:]], axis=0)
  pad = EP - E
  col = jnp.concatenate([edge_index[1], jnp.zeros((pad,), jnp.int32)])
  row = jnp.concatenate([edge_index[0], jnp.zeros((pad,), jnp.int32)])
  w = jnp.concatenate([edge_weight, jnp.zeros((pad,), jnp.float32)])
  colb = col.reshape(NBLK, NSUB, SUB)
  rowb = row.reshape(NBLK, NSUB, SUB)
  wb = w.reshape(NBLK, BLK)
  e1, e2, e3 = _prop(emb0, colb, rowb, wb)
  m = _mean(emb0.reshape(_MROWS, 128), e1.reshape(_MROWS, 128),
            e2.reshape(_MROWS, 128), e3.reshape(_MROWS, 128))
  m = m.reshape(2 * N_NODES, H)
  final = jnp.concatenate([m[:N_NODES], m[N_NODES:]], axis=1)
  return final[:N_USERS], final[N_USERS:]


# speedup vs baseline: 10.5541x; 1.3675x over previous
import functools

import jax
import jax.numpy as jnp
from jax import lax
from jax.experimental import pallas as pl
from jax.experimental.pallas import tpu as pltpu
from jax.experimental.pallas import tpu_sc as plsc

N_USERS = 25000
N_ITEMS = 25000
N_NODES = N_USERS + N_ITEMS
D = 64
H = 32
E = 800000
NL = 3

NT = 16
SUB = 128
NSUB = 4
BLK = NSUB * SUB
GPS = SUB // 16
G = 7
NSUP = 14
BPT = G * NSUP
EP = NT * BPT * BLK
NBLK = NT * BPT
RPT = N_NODES // NT


def _prop_body(emb0, colb, rowb, wb, zer, e1, e2, e3,
               colv, rowv, wv, rowsbuf, acc, gsem, ssem, isem):
  c = lax.axis_index("c")
  s = lax.axis_index("s")
  coff = c * N_NODES

  def zero_acc():
    pltpu.sync_copy(zer.at[pl.ds(s * RPT, RPT)], acc.at[pl.ds(s * RPT, RPT)])

  zero_acc()
  plsc.subcore_barrier()

  outs = (e1, e2, e3)
  for layer in range(NL):
    src = emb0 if layer == 0 else outs[layer - 1]
    dst = outs[layer]

    @pl.loop(0, NSUP)
    def _(u):
      base = s * BPT + u * G
      i1 = pltpu.async_copy(colb.at[pl.ds(base, G)], colv, isem)
      i2 = pltpu.async_copy(rowb.at[pl.ds(base, G)], rowv, isem)
      i3 = pltpu.async_copy(wb.at[pl.ds(base, G)], wv, isem)
      i1.wait()
      i2.wait()
      i3.wait()

      @pl.loop(0, G)
      def _(b):
        for j in range(NSUB):
          for q in range(SUB // 16):
            sl = (b, j, pl.ds(q * 16, 16))
            colv[sl] = colv[sl] + coff
        gd = [pltpu.async_copy(src.at[colv.at[b].at[j]], rowsbuf.at[j], gsem)
              for j in range(NSUB)]
        sd = []
        for j in range(NSUB):
          gd[j].wait()

          @pl.loop(0, GPS)
          def _(g, j=j):
            w16 = wv[b, pl.ds(j * SUB + g * 16, 16)]
            for k in range(16):
              wk = lax.broadcast_in_dim(w16[k], (16,), ())
              i = g * 16 + k
              lo = (j, i, pl.ds(0, 16))
              hi = (j, i, pl.ds(16, 16))
              rowsbuf[lo] = rowsbuf[lo] * wk
              rowsbuf[hi] = rowsbuf[hi] * wk

          sd.append(pltpu.async_copy(rowsbuf.at[j], acc.at[rowv.at[b].at[j]],
                                     ssem, add=True))
        for d_ in sd:
          d_.wait()

    plsc.subcore_barrier()
    pltpu.sync_copy(acc.at[pl.ds(s * RPT, RPT)],
                    dst.at[pl.ds(coff + s * RPT, RPT)])
    if layer < NL - 1:
      zero_acc()
    plsc.subcore_barrier()


_prop = pl.kernel(
    _prop_body,
    out_type=(jax.ShapeDtypeStruct((2 * N_NODES, H), jnp.float32),) * 3,
    mesh=plsc.VectorSubcoreMesh(core_axis_name="c", subcore_axis_name="s"),
    scratch_types=[
        pltpu.VMEM((G, NSUB, SUB), jnp.int32),
        pltpu.VMEM((G, NSUB, SUB), jnp.int32),
        pltpu.VMEM((G, BLK), jnp.float32),
        pltpu.VMEM((NSUB, SUB, H), jnp.float32),
        pltpu.VMEM_SHARED((N_NODES, H), jnp.float32),
        pltpu.SemaphoreType.DMA,
        pltpu.SemaphoreType.DMA,
        pltpu.SemaphoreType.DMA,
    ],
    compiler_params=pltpu.CompilerParams(
        use_tc_tiling_on_sc=False, needs_layout_passes=False),
)


def _mean_body(a, b, c, d, o):
  o[...] = 0.25 * (a[...] + b[...] + c[...] + d[...])


_MROWS = 2 * N_NODES * H // 128
_MB = 1000

_mean = pl.pallas_call(
    _mean_body,
    grid=(_MROWS // _MB,),
    in_specs=[pl.BlockSpec((_MB, 128), lambda i: (i, 0))] * 4,
    out_specs=pl.BlockSpec((_MB, 128), lambda i: (i, 0)),
    out_shape=jax.ShapeDtypeStruct((_MROWS, 128), jnp.float32),
)


@jax.jit
def kernel(edge_index, edge_weight, user_emb, item_emb):
  all_emb = jnp.concatenate([user_emb, item_emb], axis=0)
  emb0 = jnp.concatenate([all_emb[:, :H], all_emb[:, H:]], axis=0)
  pad = EP - E
  col = jnp.concatenate([edge_index[1], jnp.zeros((pad,), jnp.int32)])
  row = jnp.concatenate([edge_index[0], jnp.zeros((pad,), jnp.int32)])
  w = jnp.concatenate([edge_weight, jnp.zeros((pad,), jnp.float32)])
  colb = col.reshape(NBLK, NSUB, SUB)
  rowb = row.reshape(NBLK, NSUB, SUB)
  wb = w.reshape(NBLK, BLK)
  zer = jnp.zeros((N_NODES, H), jnp.float32)
  e1, e2, e3 = _prop(emb0, colb, rowb, wb, zer)
  m = _mean(emb0.reshape(_MROWS, 128), e1.reshape(_MROWS, 128),
            e2.reshape(_MROWS, 128), e3.reshape(_MROWS, 128))
  m = m.reshape(2 * N_NODES, H)
  final = jnp.concatenate([m[:N_NODES], m[N_NODES:]], axis=1)
  return final[:N_USERS], final[N_USERS:]
